# baseline (speedup 1.0000x reference)
import functools

import jax
import jax.numpy as jnp
from jax import lax
from jax.experimental import pallas as pl
from jax.experimental.pallas import tpu as pltpu
from jax.experimental.pallas import tpu_sc as plsc

N = 10000
E = 320000
D = 128
DH = D // 2

NC = 2
NS = 16
NW = NC * NS
K = 125
CH = E // (NW * K)
CH2 = E // (NS * K)
NPAD = 10240
SLAB = NPAD // NS
ZCH = SLAB // 5
NB = 4

BLK = 2000
NBLK = N // BLK

_mesh = plsc.VectorSubcoreMesh(core_axis_name="c", subcore_axis_name="s")


def _sc_degree(dst3, ones8, zeros8):
    @functools.partial(
        pl.kernel,
        out_type=jax.ShapeDtypeStruct((NC, NPAD, 8), jnp.float32),
        mesh=_mesh,
        scratch_types=[
            pltpu.VMEM((CH, K), jnp.int32),
            pltpu.VMEM((K, 8), jnp.float32),
            pltpu.VMEM((SLAB, 8), jnp.float32),
            pltpu.VMEM_SHARED((NPAD, 8), jnp.float32),
            pltpu.SemaphoreType.DMA,
        ],
        compiler_params=pltpu.CompilerParams(use_tc_tiling_on_sc=False),
    )
    def deg_kernel(dst_hbm, ones_hbm, zeros_hbm, out_hbm, dst_v, ones_v, zb_v,
                   deg_sh, sem):
        c = lax.axis_index("c")
        s = lax.axis_index("s")
        pltpu.sync_copy(dst_hbm.at[s, pl.ds(c * CH, CH)], dst_v)
        pltpu.sync_copy(ones_hbm, ones_v)
        pltpu.sync_copy(zeros_hbm, zb_v)
        pltpu.sync_copy(zb_v, deg_sh.at[pl.ds(s * SLAB, SLAB)])
        plsc.subcore_barrier()

        def fire(j, carry):
            pltpu.async_copy(ones_v, deg_sh.at[dst_v.at[j]], sem, add=True)
            return carry

        lax.fori_loop(0, CH, fire, 0)

        def drain(j, carry):
            pltpu.make_async_copy(ones_v, deg_sh.at[dst_v.at[j]], sem).wait()
            return carry

        lax.fori_loop(0, CH, drain, 0)
        plsc.subcore_barrier()
        pltpu.sync_copy(deg_sh.at[pl.ds(s * SLAB, SLAB)],
                        out_hbm.at[c, pl.ds(s * SLAB, SLAB)])

    return deg_kernel(dst3, ones8, zeros8)


def _tc_matmul(x, W):
    def body(x_ref, w_ref, h_ref):
        h = jnp.dot(x_ref[...], w_ref[...], preferred_element_type=jnp.float32)
        h_ref[0, :, :] = h[:, :DH]
        h_ref[1, :, :] = h[:, DH:]

    return pl.pallas_call(
        body,
        grid=(NBLK,),
        in_specs=[
            pl.BlockSpec((BLK, D), lambda i: (i, 0)),
            pl.BlockSpec((D, D), lambda i: (0, 0)),
        ],
        out_specs=pl.BlockSpec((NC, BLK, DH), lambda i: (0, i, 0)),
        out_shape=jax.ShapeDtypeStruct((NC, N, DH), jnp.float32),
    )(x, W)


def _tc_scale(h2, deg_p):
    def body(h_ref, deg_ref, g_ref):
        d = deg_ref[...]
        dinv = lax.rsqrt(d[0, :, 0:1] + d[1, :, 0:1] + 1.0)
        g_ref[0, :, :] = h_ref[0] * dinv
        g_ref[1, :, :] = h_ref[1] * dinv

    return pl.pallas_call(
        body,
        grid=(NBLK,),
        in_specs=[
            pl.BlockSpec((NC, BLK, DH), lambda i: (0, i, 0)),
            pl.BlockSpec((NC, BLK, 8), lambda i: (0, i, 0)),
        ],
        out_specs=pl.BlockSpec((NC, BLK, DH), lambda i: (0, i, 0)),
        out_shape=jax.ShapeDtypeStruct((NC, N, DH), jnp.float32),
    )(h2, deg_p)


def _sc_scatter(g2, srcq, dst3, zerosD):
    @functools.partial(
        pl.kernel,
        out_type=jax.ShapeDtypeStruct((NC, NPAD, DH), jnp.float32),
        mesh=_mesh,
        scratch_types=[
            pltpu.VMEM((CH2, K), jnp.int32),
            pltpu.VMEM((CH2, K), jnp.int32),
            [pltpu.VMEM((K, DH), jnp.float32)] * NB,
            pltpu.VMEM((ZCH, DH), jnp.float32),
            pltpu.VMEM_SHARED((NPAD, DH), jnp.float32),
            [pltpu.SemaphoreType.DMA] * NB,
            [pltpu.SemaphoreType.DMA] * NB,
        ],
        compiler_params=pltpu.CompilerParams(use_tc_tiling_on_sc=False),
    )
    def scat_kernel(g_hbm, src_hbm, dst_hbm, zeros_hbm, out_hbm,
                    src_v, dst_v, rows, zb_v, acc_sh, gsem, ssem):
        c = lax.axis_index("c")
        s = lax.axis_index("s")
        gc = g_hbm.at[c]
        pltpu.sync_copy(src_hbm.at[s], src_v)
        pltpu.sync_copy(dst_hbm.at[s], dst_v)
        pltpu.sync_copy(zeros_hbm, zb_v)
        for k in range(SLAB // ZCH):
            pltpu.sync_copy(zb_v, acc_sh.at[pl.ds(s * SLAB + k * ZCH, ZCH)])
        plsc.subcore_barrier()

        for b in range(NB):
            pltpu.async_copy(gc.at[src_v.at[b]], rows[b], gsem[b])

        def body(block, carry):
            base = block * NB
            for b in range(NB):
                j = base + b
                pltpu.make_async_copy(gc.at[src_v.at[j]], rows[b],
                                      gsem[b]).wait()
                pltpu.async_copy(rows[b], acc_sh.at[dst_v.at[j]], ssem[b],
                                 add=True)
            for b in range(NB):
                j = base + b
                pltpu.make_async_copy(rows[b], acc_sh.at[dst_v.at[j]],
                                      ssem[b]).wait()

                @pl.when(j + NB < CH2)
                def _():
                    pltpu.async_copy(gc.at[src_v.at[j + NB]], rows[b],
                                     gsem[b])

            return carry

        lax.fori_loop(0, CH2 // NB, body, 0)
        plsc.subcore_barrier()
        pltpu.sync_copy(acc_sh.at[pl.ds(s * SLAB, SLAB)],
                        out_hbm.at[c, pl.ds(s * SLAB, SLAB)])

    return scat_kernel(g2, srcq, dst3, zerosD)


def _tc_finish(acc_p, g2, deg_p, b):
    def body(acc_ref, g_ref, deg_ref, b_ref, o_ref):
        d = deg_ref[...]
        dinv = lax.rsqrt(d[0, :, 0:1] + d[1, :, 0:1] + 1.0)
        a = jnp.concatenate([acc_ref[0] + g_ref[0], acc_ref[1] + g_ref[1]],
                            axis=-1)
        o_ref[...] = jnp.maximum(a * dinv + b_ref[...], 0.0)

    return pl.pallas_call(
        body,
        grid=(NBLK,),
        in_specs=[
            pl.BlockSpec((NC, BLK, DH), lambda i: (0, i, 0)),
            pl.BlockSpec((NC, BLK, DH), lambda i: (0, i, 0)),
            pl.BlockSpec((NC, BLK, 8), lambda i: (0, i, 0)),
            pl.BlockSpec((1, D), lambda i: (0, 0)),
        ],
        out_specs=pl.BlockSpec((BLK, D), lambda i: (i, 0)),
        out_shape=jax.ShapeDtypeStruct((N, D), jnp.float32),
    )(acc_p, g2, deg_p, b.reshape(1, D))


def kernel(x, edge_index, W, b):
    src2 = edge_index[0].reshape(NS, CH2, K)
    dstq = edge_index[1].reshape(NS, CH2, K)
    ones8 = jnp.ones((K, 8), jnp.float32)
    zeros8 = jnp.zeros((SLAB, 8), jnp.float32)
    zerosD = jnp.zeros((ZCH, DH), jnp.float32)

    deg_p = _sc_degree(dstq, ones8, zeros8)
    h2 = _tc_matmul(x, W)
    g2 = _tc_scale(h2, deg_p)
    acc_p = _sc_scatter(g2, src2, dstq, zerosD)
    return _tc_finish(acc_p, g2, deg_p, b)


# speedup vs baseline: 1.0592x; 1.0592x over previous
import functools

import jax
import jax.numpy as jnp
from jax import lax
from jax.experimental import pallas as pl
from jax.experimental.pallas import tpu as pltpu
from jax.experimental.pallas import tpu_sc as plsc

N = 10000
E = 320000
D = 128
DH = D // 2

NC = 2
NS = 16
NW = NC * NS
K = 125
CH = E // (NW * K)
CH2 = E // (NS * K)
NPAD = 10240
SLAB = NPAD // NS
ZCH = SLAB // 5
NB = 4

BLK = 2000
NBLK = N // BLK

_mesh = plsc.VectorSubcoreMesh(core_axis_name="c", subcore_axis_name="s")


def _sc_degree(ei4, ones8, zeros8):
    @functools.partial(
        pl.kernel,
        out_type=jax.ShapeDtypeStruct((NC, NPAD, 8), jnp.float32),
        mesh=_mesh,
        scratch_types=[
            pltpu.VMEM((CH, K), jnp.int32),
            pltpu.VMEM((K, 8), jnp.float32),
            pltpu.VMEM((SLAB, 8), jnp.float32),
            pltpu.VMEM_SHARED((NPAD, 8), jnp.float32),
            pltpu.SemaphoreType.DMA,
        ],
        compiler_params=pltpu.CompilerParams(use_tc_tiling_on_sc=False),
    )
    def deg_kernel(ei_hbm, ones_hbm, zeros_hbm, out_hbm, dst_v, ones_v, zb_v,
                   deg_sh, sem):
        c = lax.axis_index("c")
        s = lax.axis_index("s")
        pltpu.sync_copy(ei_hbm.at[1, s, pl.ds(c * CH, CH)], dst_v)
        pltpu.sync_copy(ones_hbm, ones_v)
        pltpu.sync_copy(zeros_hbm, zb_v)
        pltpu.sync_copy(zb_v, deg_sh.at[pl.ds(s * SLAB, SLAB)])
        plsc.subcore_barrier()

        def fire(j, carry):
            pltpu.async_copy(ones_v, deg_sh.at[dst_v.at[j]], sem, add=True)
            return carry

        lax.fori_loop(0, CH, fire, 0)

        def drain(j, carry):
            pltpu.make_async_copy(ones_v, deg_sh.at[dst_v.at[j]], sem).wait()
            return carry

        lax.fori_loop(0, CH, drain, 0)
        plsc.subcore_barrier()
        pltpu.sync_copy(deg_sh.at[pl.ds(s * SLAB, SLAB)],
                        out_hbm.at[c, pl.ds(s * SLAB, SLAB)])

    return deg_kernel(ei4, ones8, zeros8)


def _tc_transform(x, W, deg_p):
    def body(x_ref, w_ref, deg_ref, g_ref):
        h = jnp.dot(x_ref[...], w_ref[...], preferred_element_type=jnp.float32)
        d = deg_ref[...]
        dinv = lax.rsqrt(d[0, :, 0:1] + d[1, :, 0:1] + 1.0)
        g = h * dinv
        g_ref[0, :, :] = g[:, :DH]
        g_ref[1, :, :] = g[:, DH:]

    return pl.pallas_call(
        body,
        grid=(NBLK,),
        in_specs=[
            pl.BlockSpec((BLK, D), lambda i: (i, 0)),
            pl.BlockSpec((D, D), lambda i: (0, 0)),
            pl.BlockSpec((NC, BLK, 8), lambda i: (0, i, 0)),
        ],
        out_specs=pl.BlockSpec((NC, BLK, DH), lambda i: (0, i, 0)),
        out_shape=jax.ShapeDtypeStruct((NC, N, DH), jnp.float32),
    )(x, W, deg_p)


def _sc_scatter(g2, ei4, zerosD):
    @functools.partial(
        pl.kernel,
        out_type=jax.ShapeDtypeStruct((NC, NPAD, DH), jnp.float32),
        mesh=_mesh,
        scratch_types=[
            pltpu.VMEM((CH2, K), jnp.int32),
            pltpu.VMEM((CH2, K), jnp.int32),
            [pltpu.VMEM((K, DH), jnp.float32)] * NB,
            pltpu.VMEM((ZCH, DH), jnp.float32),
            pltpu.VMEM_SHARED((NPAD, DH), jnp.float32),
            [pltpu.SemaphoreType.DMA] * NB,
            [pltpu.SemaphoreType.DMA] * NB,
        ],
        compiler_params=pltpu.CompilerParams(use_tc_tiling_on_sc=False),
    )
    def scat_kernel(g_hbm, ei_hbm, zeros_hbm, out_hbm,
                    src_v, dst_v, rows, zb_v, acc_sh, gsem, ssem):
        c = lax.axis_index("c")
        s = lax.axis_index("s")
        gc = g_hbm.at[c]
        pltpu.sync_copy(ei_hbm.at[0, s], src_v)
        pltpu.sync_copy(ei_hbm.at[1, s], dst_v)
        pltpu.sync_copy(zeros_hbm, zb_v)
        for k in range(SLAB // ZCH):
            pltpu.sync_copy(zb_v, acc_sh.at[pl.ds(s * SLAB + k * ZCH, ZCH)])
        plsc.subcore_barrier()

        for b in range(NB):
            pltpu.async_copy(gc.at[src_v.at[b]], rows[b], gsem[b])

        def body(block, carry):
            base = block * NB
            for b in range(NB):
                j = base + b
                pltpu.make_async_copy(gc.at[src_v.at[j]], rows[b],
                                      gsem[b]).wait()
                pltpu.async_copy(rows[b], acc_sh.at[dst_v.at[j]], ssem[b],
                                 add=True)
            for b in range(NB):
                j = base + b
                pltpu.make_async_copy(rows[b], acc_sh.at[dst_v.at[j]],
                                      ssem[b]).wait()

                @pl.when(j + NB < CH2)
                def _():
                    pltpu.async_copy(gc.at[src_v.at[j + NB]], rows[b],
                                     gsem[b])

            return carry

        lax.fori_loop(0, CH2 // NB, body, 0)
        plsc.subcore_barrier()
        pltpu.sync_copy(acc_sh.at[pl.ds(s * SLAB, SLAB)],
                        out_hbm.at[c, pl.ds(s * SLAB, SLAB)])

    return scat_kernel(g2, ei4, zerosD)


def _tc_finish(acc_p, g2, deg_p, b):
    def body(acc_ref, g_ref, deg_ref, b_ref, o_ref):
        d = deg_ref[...]
        dinv = lax.rsqrt(d[0, :, 0:1] + d[1, :, 0:1] + 1.0)
        a = jnp.concatenate([acc_ref[0] + g_ref[0], acc_ref[1] + g_ref[1]],
                            axis=-1)
        o_ref[...] = jnp.maximum(a * dinv + b_ref[...], 0.0)

    return pl.pallas_call(
        body,
        grid=(NBLK,),
        in_specs=[
            pl.BlockSpec((NC, BLK, DH), lambda i: (0, i, 0)),
            pl.BlockSpec((NC, BLK, DH), lambda i: (0, i, 0)),
            pl.BlockSpec((NC, BLK, 8), lambda i: (0, i, 0)),
            pl.BlockSpec((1, D), lambda i: (0, 0)),
        ],
        out_specs=pl.BlockSpec((BLK, D), lambda i: (i, 0)),
        out_shape=jax.ShapeDtypeStruct((N, D), jnp.float32),
    )(acc_p, g2, deg_p, b.reshape(1, D))


def kernel(x, edge_index, W, b):
    ei4 = edge_index.reshape(2, NS, CH2, K)
    ones8 = jnp.ones((K, 8), jnp.float32)
    zeros8 = jnp.zeros((SLAB, 8), jnp.float32)
    zerosD = jnp.zeros((ZCH, DH), jnp.float32)

    deg_p = _sc_degree(ei4, ones8, zeros8)
    g2 = _tc_transform(x, W, deg_p)
    acc_p = _sc_scatter(g2, ei4, zerosD)
    return _tc_finish(acc_p, g2, deg_p, b)
